# Initial kernel scaffold; baseline (speedup 1.0000x reference)
import jax
import jax.numpy as jnp
from jax.experimental import pallas as pl


def kernel(doc, table):
    raise NotImplementedError("write your pallas kernel here")


# speedup vs baseline: 1.4776x; 1.4776x over previous
import functools

import jax
import jax.numpy as jnp
from jax import lax
from jax.experimental import pallas as pl
from jax.experimental.pallas import tpu as pltpu
from jax.experimental.pallas import tpu_sc as plsc

_VOCAB = 1000000
_D = 32
_BATCH = 4096
_HIST = 200
_N = _BATCH * _HIST
_NC, _NS = 2, 16
_NW = _NC * _NS
_PER_W = _N // _NW
_C = 1600
_NIT = _PER_W // _C


def _make_gather():
  mesh = plsc.VectorSubcoreMesh(
      core_axis_name="c", subcore_axis_name="s",
      num_cores=_NC, num_subcores=_NS)

  @functools.partial(
      pl.kernel,
      mesh=mesh,
      out_type=jax.ShapeDtypeStruct((_N, _D), jnp.float32),
      scratch_types=[
          pltpu.VMEM((_C,), jnp.int32),
          pltpu.VMEM((_C, _D), jnp.float32),
          pltpu.SemaphoreType.DMA,
      ],
      compiler_params=pltpu.CompilerParams(use_tc_tiling_on_sc=False),
  )
  def gather(doc_hbm, table_hbm, out_hbm, idx_v, rows_v, sem):
    wid = lax.axis_index("s") * _NC + lax.axis_index("c")
    base = wid * _PER_W

    def body(i, carry):
      off = base + i * _C
      pltpu.sync_copy(doc_hbm.at[pl.ds(off, _C)], idx_v)
      pltpu.async_copy(table_hbm.at[idx_v], rows_v, sem).wait()
      pltpu.sync_copy(rows_v, out_hbm.at[pl.ds(off, _C)])
      return carry

    lax.fori_loop(0, _NIT, body, 0)

  return gather


_gather = _make_gather()


def kernel(doc, table):
  flat = doc.reshape(_N)
  out = _gather(flat, table)
  return out.reshape(_BATCH, _HIST, _D)


# speedup vs baseline: 1.4936x; 1.0108x over previous
import functools

import jax
import jax.numpy as jnp
from jax import lax
from jax.experimental import pallas as pl
from jax.experimental.pallas import tpu as pltpu
from jax.experimental.pallas import tpu_sc as plsc

_VOCAB = 1000000
_D = 32
_BATCH = 4096
_HIST = 200
_N = _BATCH * _HIST
_NC, _NS = 2, 16
_NW = _NC * _NS
_PER_W = _N // _NW
_C = 1600
_NBUF = 2
_CG = _C * _NBUF
_G = _PER_W // _CG


def _make_gather():
  mesh = plsc.VectorSubcoreMesh(
      core_axis_name="c", subcore_axis_name="s",
      num_cores=_NC, num_subcores=_NS)

  @functools.partial(
      pl.kernel,
      mesh=mesh,
      out_type=jax.ShapeDtypeStruct((_N, _D), jnp.float32),
      scratch_types=[
          pltpu.VMEM((_NBUF, _C), jnp.int32),
          pltpu.VMEM((_NBUF, _C, _D), jnp.float32),
          [pltpu.SemaphoreType.DMA] * _NBUF,
          [pltpu.SemaphoreType.DMA] * _NBUF,
          [pltpu.SemaphoreType.DMA] * _NBUF,
      ],
      compiler_params=pltpu.CompilerParams(use_tc_tiling_on_sc=False),
  )
  def gather(doc_hbm, table_hbm, out_hbm, idx_v, rows_v, isems, gsems, wsems):
    wid = lax.axis_index("s") * _NC + lax.axis_index("c")
    base = wid * _PER_W

    for b in range(_NBUF):
      pltpu.async_copy(
          doc_hbm.at[pl.ds(base + b * _C, _C)], idx_v.at[b], isems[b])

    def body(g, carry):
      off0 = base + g * _CG
      for b in range(_NBUF):
        @pl.when(g > 0)
        def _():
          pltpu.make_async_copy(
              rows_v.at[b], out_hbm.at[pl.ds(base, _C)], wsems[b]).wait()
        pltpu.make_async_copy(
            doc_hbm.at[pl.ds(base, _C)], idx_v.at[b], isems[b]).wait()
        pltpu.async_copy(table_hbm.at[idx_v.at[b]], rows_v.at[b], gsems[b])
      for b in range(_NBUF):
        pltpu.make_async_copy(
            table_hbm.at[idx_v.at[b]], rows_v.at[b], gsems[b]).wait()
        pltpu.async_copy(
            rows_v.at[b], out_hbm.at[pl.ds(off0 + b * _C, _C)], wsems[b])

        @pl.when(g + 1 < _G)
        def _():
          pltpu.async_copy(
              doc_hbm.at[pl.ds(off0 + _CG + b * _C, _C)],
              idx_v.at[b], isems[b])
      return carry

    lax.fori_loop(0, _G, body, 0)

    for b in range(_NBUF):
      pltpu.make_async_copy(
          rows_v.at[b], out_hbm.at[pl.ds(base, _C)], wsems[b]).wait()

  return gather


_gather = _make_gather()


def kernel(doc, table):
  flat = doc.reshape(_N)
  out = _gather(flat, table)
  return out.reshape(_BATCH, _HIST, _D)
